# baseline (speedup 1.0000x reference)
import jax
import jax.numpy as jnp
from jax.experimental import pallas as pl
from jax.experimental.pallas import tpu as pltpu

HIDDEN_DIM = 2048
NUM_EXPERTS = 8
EXPERT_DIM = 4096

M_BLK = 1024
E_BLK = 1024
N_E = EXPERT_DIM // E_BLK

_DN_T = (((1,), (1,)), ((), ()))


def _moe_body(x_ref, wr_ref, wg_ref, wu_ref, wd_ref, out_ref, s_ref):
    e = pl.program_id(1)
    xb = x_ref[...]

    @pl.when(e == 0)
    def _router():
        logits = jax.lax.dot_general(
            xb, wr_ref[...], _DN_T,
            preferred_element_type=jnp.float32)
        neg_inf = jnp.float32(-jnp.inf)
        m1 = jnp.max(logits, axis=1, keepdims=True)
        eq = logits == m1
        cnt = jnp.sum(eq.astype(jnp.float32), axis=1, keepdims=True)
        m2 = jnp.max(jnp.where(eq, neg_inf, logits), axis=1, keepdims=True)
        l2 = jnp.where(cnt >= 2.0, m1, m2)
        z = jnp.sum(jnp.exp(logits - m1), axis=1, keepdims=True)
        s_ref[...] = (1.0 + jnp.exp(l2 - m1)) / z

    s = s_ref[...]
    gate = jax.lax.dot_general(
        xb, wg_ref[...], _DN_T, preferred_element_type=jnp.float32)
    up = jax.lax.dot_general(
        xb, wu_ref[...], _DN_T, preferred_element_type=jnp.float32)
    act = (gate * jax.nn.sigmoid(gate) * up * s).astype(jnp.bfloat16)
    half = E_BLK // 2
    p_a = jax.lax.dot_general(
        act[:, :half], wd_ref[:, :half], _DN_T,
        preferred_element_type=jnp.float32)
    out_ref[...] = jnp.where(e > 0, out_ref[...], 0.0) + p_a
    p_b = jax.lax.dot_general(
        act[:, half:], wd_ref[:, half:], _DN_T,
        preferred_element_type=jnp.float32)
    out_ref[...] += p_b


def kernel(x, W_router, W_gate, W_up, W_down):
    orig_shape = x.shape
    tokens = orig_shape[0] * orig_shape[1]
    xf = x.reshape(tokens, HIDDEN_DIM).astype(jnp.bfloat16)
    wr = W_router.astype(jnp.bfloat16)
    wg = W_gate.astype(jnp.bfloat16)
    wu = W_up.astype(jnp.bfloat16)
    wd = W_down.astype(jnp.bfloat16)

    n_m = tokens // M_BLK

    out = pl.pallas_call(
        _moe_body,
        grid=(n_m, N_E),
        in_specs=[
            pl.BlockSpec((M_BLK, HIDDEN_DIM), lambda m, e: (m, 0)),
            pl.BlockSpec((NUM_EXPERTS, HIDDEN_DIM), lambda m, e: (0, 0)),
            pl.BlockSpec((E_BLK, HIDDEN_DIM), lambda m, e: (e, 0)),
            pl.BlockSpec((E_BLK, HIDDEN_DIM), lambda m, e: (e, 0)),
            pl.BlockSpec((HIDDEN_DIM, E_BLK), lambda m, e: (0, e)),
        ],
        out_specs=pl.BlockSpec((M_BLK, HIDDEN_DIM), lambda m, e: (m, 0)),
        out_shape=jax.ShapeDtypeStruct((tokens, HIDDEN_DIM), jnp.float32),
        scratch_shapes=[
            pltpu.VMEM((M_BLK, 1), jnp.float32),
        ],
        compiler_params=pltpu.CompilerParams(
            dimension_semantics=("parallel", "arbitrary")),
    )(xf, wr, wg, wu, wd)
    return out.reshape(orig_shape)


# speedup vs baseline: 1.0327x; 1.0327x over previous
import jax
import jax.numpy as jnp
from jax.experimental import pallas as pl
from jax.experimental.pallas import tpu as pltpu

HIDDEN_DIM = 2048
NUM_EXPERTS = 8
EXPERT_DIM = 4096

M_BLK = 512
E_BLK = 1024
N_E = EXPERT_DIM // E_BLK

_DN_T = (((1,), (1,)), ((), ()))


def _moe_body(x_ref, wr_ref, wg_ref, wu_ref, wd_ref, out_ref, s_ref):
    e = pl.program_id(1)
    xb = x_ref[...].astype(jnp.bfloat16)

    @pl.when(e == 0)
    def _router():
        logits = jax.lax.dot_general(
            xb, wr_ref[...], _DN_T,
            preferred_element_type=jnp.float32)
        neg_inf = jnp.float32(-jnp.inf)
        m1 = jnp.max(logits, axis=1, keepdims=True)
        eq = logits == m1
        cnt = jnp.sum(eq.astype(jnp.float32), axis=1, keepdims=True)
        m2 = jnp.max(jnp.where(eq, neg_inf, logits), axis=1, keepdims=True)
        l2 = jnp.where(cnt >= 2.0, m1, m2)
        z = jnp.sum(jnp.exp(logits - m1), axis=1, keepdims=True)
        s_ref[...] = (1.0 + jnp.exp(l2 - m1)) / z

    s = s_ref[...]
    gate = jax.lax.dot_general(
        xb, wg_ref[...], _DN_T, preferred_element_type=jnp.float32)
    up = jax.lax.dot_general(
        xb, wu_ref[...], _DN_T, preferred_element_type=jnp.float32)
    act = (gate * jax.nn.sigmoid(gate) * up * s).astype(jnp.bfloat16)
    half = E_BLK // 2
    p_a = jax.lax.dot_general(
        act[:, :half], wd_ref[:, :half], _DN_T,
        preferred_element_type=jnp.float32)
    out_ref[...] = jnp.where(e > 0, out_ref[...], 0.0) + p_a
    p_b = jax.lax.dot_general(
        act[:, half:], wd_ref[:, half:], _DN_T,
        preferred_element_type=jnp.float32)
    out_ref[...] += p_b


def kernel(x, W_router, W_gate, W_up, W_down):
    orig_shape = x.shape
    tokens = orig_shape[0] * orig_shape[1]
    xf = x.reshape(tokens, HIDDEN_DIM)
    wr = W_router.astype(jnp.bfloat16)
    wg = W_gate.astype(jnp.bfloat16)
    wu = W_up.astype(jnp.bfloat16)
    wd = W_down.astype(jnp.bfloat16)

    n_m = tokens // M_BLK

    out = pl.pallas_call(
        _moe_body,
        grid=(n_m, N_E),
        in_specs=[
            pl.BlockSpec((M_BLK, HIDDEN_DIM), lambda m, e: (m, 0)),
            pl.BlockSpec((NUM_EXPERTS, HIDDEN_DIM), lambda m, e: (0, 0)),
            pl.BlockSpec((E_BLK, HIDDEN_DIM), lambda m, e: (e, 0)),
            pl.BlockSpec((E_BLK, HIDDEN_DIM), lambda m, e: (e, 0)),
            pl.BlockSpec((HIDDEN_DIM, E_BLK), lambda m, e: (0, e)),
        ],
        out_specs=pl.BlockSpec((M_BLK, HIDDEN_DIM), lambda m, e: (m, 0)),
        out_shape=jax.ShapeDtypeStruct((tokens, HIDDEN_DIM), jnp.float32),
        scratch_shapes=[
            pltpu.VMEM((M_BLK, 1), jnp.float32),
        ],
        compiler_params=pltpu.CompilerParams(
            dimension_semantics=("parallel", "arbitrary")),
    )(xf, wr, wg, wu, wd)
    return out.reshape(orig_shape)
